# Initial kernel scaffold; baseline (speedup 1.0000x reference)
import jax
import jax.numpy as jnp
from jax.experimental import pallas as pl


def kernel(x, codebooks):
    raise NotImplementedError("write your pallas kernel here")


# speedup vs baseline: 1.6808x; 1.6808x over previous
import functools

import jax
import jax.numpy as jnp
from jax import lax
from jax.experimental import pallas as pl
from jax.experimental.pallas import tpu as pltpu
from jax.experimental.pallas import tpu_sc as plsc

B, T, D = 32, 576, 256
N = B * T
K = 8192
TM = 512
NT = N // TM

NC, NS = 2, 16
NW = NC * NS
ROWS_W = N // NW
CH = 96
NCH = ROWS_W // CH


def _tc_stage(r_prev, q_prev, emb, b_row, x_for_partial):
    with_prev = q_prev is not None
    with_partial = x_for_partial is not None
    emit_r = with_prev or with_partial

    def body(*refs):
        i = 0
        r_ref = refs[i]; i += 1
        if with_prev:
            q_ref = refs[i]; i += 1
        e_ref = refs[i]; i += 1
        b_ref = refs[i]; i += 1
        if with_partial:
            x_ref = refs[i]; i += 1
        codes_ref = refs[i]; i += 1
        if emit_r:
            rout_ref = refs[i]; i += 1

        r = r_ref[...]
        if with_prev:
            r = r - q_ref[...]
        rowsq = jnp.sum(r * r, axis=1, keepdims=True)
        mm = lax.dot_general(r, e_ref[...], (((1,), (1,)), ((), ())),
                             preferred_element_type=jnp.float32)
        d2 = rowsq - 2.0 * mm + b_ref[...]
        mins = jnp.min(d2, axis=1, keepdims=True)
        iot = lax.broadcasted_iota(jnp.int32, d2.shape, 1)
        idx = jnp.min(jnp.where(d2 == mins, iot, K), axis=1)
        codes_ref[...] = idx.reshape(1, 1, TM)
        if with_partial:
            rout_ref[...] = (1.0 + x_ref[...]) - r
        elif emit_r:
            rout_ref[...] = r

    row_spec = pl.BlockSpec((TM, D), lambda i: (i, 0))
    in_specs = [row_spec]
    inputs = [r_prev]
    if with_prev:
        in_specs.append(row_spec)
        inputs.append(q_prev)
    in_specs.append(pl.BlockSpec((K, D), lambda i: (0, 0)))
    inputs.append(emb)
    in_specs.append(pl.BlockSpec((1, K), lambda i: (0, 0)))
    inputs.append(b_row)
    if with_partial:
        in_specs.append(row_spec)
        inputs.append(x_for_partial)

    out_shapes = [jax.ShapeDtypeStruct((NT, 1, TM), jnp.int32)]
    out_specs = [pl.BlockSpec((1, 1, TM), lambda i: (i, 0, 0))]
    if emit_r:
        out_shapes.append(jax.ShapeDtypeStruct((N, D), jnp.float32))
        out_specs.append(row_spec)

    out = pl.pallas_call(
        body,
        grid=(NT,),
        in_specs=in_specs,
        out_specs=out_specs,
        out_shape=out_shapes,
        compiler_params=pltpu.CompilerParams(
            dimension_semantics=("parallel",)),
    )(*inputs)
    return out if emit_r else (out[0], None)


@functools.partial(
    pl.kernel,
    mesh=plsc.VectorSubcoreMesh(core_axis_name="c", subcore_axis_name="s"),
    out_type=jax.ShapeDtypeStruct((N, D), jnp.float32),
    scratch_types=[
        pltpu.VMEM((NCH, CH), jnp.int32),
        pltpu.VMEM((CH, D), jnp.float32),
        pltpu.SemaphoreType.DMA,
    ],
)
def _sc_gather(emb_hbm, codes_hbm, q_hbm, idx_v, rows_v, sem):
    wid = lax.axis_index("s") * NC + lax.axis_index("c")
    pltpu.sync_copy(codes_hbm.at[wid], idx_v)
    base = wid * ROWS_W
    for c in range(NCH):
        pltpu.async_copy(emb_hbm.at[idx_v.at[c]], rows_v, sem).wait()
        pltpu.sync_copy(rows_v, q_hbm.at[pl.ds(base + c * CH, CH)])


def _fin(x, r3, q4):
    def body(x_ref, r_ref, q_ref, o_ref):
        o_ref[...] = (1.0 + x_ref[...]) - (r_ref[...] - q_ref[...])

    spec = pl.BlockSpec((TM, D), lambda i: (i, 0))
    return pl.pallas_call(
        body,
        grid=(NT,),
        in_specs=[spec, spec, spec],
        out_specs=spec,
        out_shape=jax.ShapeDtypeStruct((N, D), jnp.float32),
        compiler_params=pltpu.CompilerParams(
            dimension_semantics=("parallel",)),
    )(x, r3, q4)


def kernel(x, codebooks):
    xf = x.reshape(N, D)
    bnorm = jnp.sum(codebooks * codebooks, axis=-1)

    def stage(r_prev, q_prev, s, x_for_partial=None):
        codes, r_out = _tc_stage(r_prev, q_prev, codebooks[s],
                                 bnorm[s].reshape(1, K), x_for_partial)
        codes32 = codes.reshape(NW, NCH, CH)
        q = _sc_gather(codebooks[s], codes32)
        return q, r_out

    q1, _ = stage(xf, None, 0)
    q2, r1 = stage(xf, q1, 1)
    q3, r2 = stage(r1, q2, 2)
    q4, r3 = stage(r2, q3, 3, x_for_partial=None)
    out = _fin(xf, r3, q4)
    return out.reshape(B, T, D)


# speedup vs baseline: 1.6908x; 1.0060x over previous
import functools

import jax
import jax.numpy as jnp
from jax import lax
from jax.experimental import pallas as pl
from jax.experimental.pallas import tpu as pltpu
from jax.experimental.pallas import tpu_sc as plsc

B, T, D = 32, 576, 256
N = B * T
K = 8192
TM = 512
NT = N // TM
KB = 2048

NC, NS = 2, 16
NW = NC * NS
ROWS_W = N // NW
CH = 96
NCH = ROWS_W // CH


def _tc_stage(r_prev, q_prev, emb, b_row, x_for_partial):
    with_prev = q_prev is not None
    with_partial = x_for_partial is not None
    emit_r = with_prev or with_partial

    def body(*refs):
        i = 0
        r_ref = refs[i]; i += 1
        if with_prev:
            q_ref = refs[i]; i += 1
        e_ref = refs[i]; i += 1
        b_ref = refs[i]; i += 1
        if with_partial:
            x_ref = refs[i]; i += 1
        codes_ref = refs[i]; i += 1
        if emit_r:
            rout_ref = refs[i]; i += 1

        r = r_ref[...]
        if with_prev:
            r = r - q_ref[...]
        rowsq = jnp.sum(r * r, axis=1, keepdims=True)
        rm2 = r * (-2.0)
        iotaf = lax.broadcasted_iota(jnp.int32, (TM, KB), 1).astype(jnp.float32)
        m_run = None
        for kb in range(K // KB):
            e_blk = e_ref[pl.ds(kb * KB, KB), :]
            mm = lax.dot_general(rm2, e_blk, (((1,), (1,)), ((), ())),
                                 preferred_element_type=jnp.float32)
            d2 = (rowsq + mm) + b_ref[:, pl.ds(kb * KB, KB)]
            mins = jnp.min(d2, axis=1, keepdims=True)
            idxf = jnp.min(
                jnp.where(d2 == mins, iotaf + (kb * KB), jnp.inf),
                axis=1, keepdims=True)
            if m_run is None:
                m_run, i_run = mins, idxf
            else:
                upd = mins < m_run
                m_run = jnp.where(upd, mins, m_run)
                i_run = jnp.where(upd, idxf, i_run)
        idx = i_run[:, 0].astype(jnp.int32)
        codes_ref[...] = idx.reshape(1, 1, TM)
        if with_partial:
            rout_ref[...] = (1.0 + x_ref[...]) - r
        elif emit_r:
            rout_ref[...] = r

    row_spec = pl.BlockSpec((TM, D), lambda i: (i, 0))
    in_specs = [row_spec]
    inputs = [r_prev]
    if with_prev:
        in_specs.append(row_spec)
        inputs.append(q_prev)
    in_specs.append(pl.BlockSpec((K, D), lambda i: (0, 0)))
    inputs.append(emb)
    in_specs.append(pl.BlockSpec((1, K), lambda i: (0, 0)))
    inputs.append(b_row)
    if with_partial:
        in_specs.append(row_spec)
        inputs.append(x_for_partial)

    out_shapes = [jax.ShapeDtypeStruct((NT, 1, TM), jnp.int32)]
    out_specs = [pl.BlockSpec((1, 1, TM), lambda i: (i, 0, 0))]
    if emit_r:
        out_shapes.append(jax.ShapeDtypeStruct((N, D), jnp.float32))
        out_specs.append(row_spec)

    out = pl.pallas_call(
        body,
        grid=(NT,),
        in_specs=in_specs,
        out_specs=out_specs,
        out_shape=out_shapes,
        compiler_params=pltpu.CompilerParams(
            dimension_semantics=("parallel",)),
    )(*inputs)
    return out if emit_r else (out[0], None)


@functools.partial(
    pl.kernel,
    mesh=plsc.VectorSubcoreMesh(core_axis_name="c", subcore_axis_name="s"),
    out_type=jax.ShapeDtypeStruct((N, D), jnp.float32),
    scratch_types=[
        pltpu.VMEM((NCH, CH), jnp.int32),
        pltpu.VMEM((CH, D), jnp.float32),
        pltpu.SemaphoreType.DMA,
    ],
)
def _sc_gather(emb_hbm, codes_hbm, q_hbm, idx_v, rows_v, sem):
    wid = lax.axis_index("s") * NC + lax.axis_index("c")
    pltpu.sync_copy(codes_hbm.at[wid], idx_v)
    base = wid * ROWS_W
    for c in range(NCH):
        pltpu.async_copy(emb_hbm.at[idx_v.at[c]], rows_v, sem).wait()
        pltpu.sync_copy(rows_v, q_hbm.at[pl.ds(base + c * CH, CH)])


def _fin(x, r3, q4):
    def body(x_ref, r_ref, q_ref, o_ref):
        o_ref[...] = (1.0 + x_ref[...]) - (r_ref[...] - q_ref[...])

    spec = pl.BlockSpec((TM, D), lambda i: (i, 0))
    return pl.pallas_call(
        body,
        grid=(NT,),
        in_specs=[spec, spec, spec],
        out_specs=spec,
        out_shape=jax.ShapeDtypeStruct((N, D), jnp.float32),
        compiler_params=pltpu.CompilerParams(
            dimension_semantics=("parallel",)),
    )(x, r3, q4)


def kernel(x, codebooks):
    xf = x.reshape(N, D)
    bnorm = jnp.sum(codebooks * codebooks, axis=-1)

    def stage(r_prev, q_prev, s, x_for_partial=None):
        codes, r_out = _tc_stage(r_prev, q_prev, codebooks[s],
                                 bnorm[s].reshape(1, K), x_for_partial)
        codes32 = codes.reshape(NW, NCH, CH)
        q = _sc_gather(codebooks[s], codes32)
        return q, r_out

    q1, _ = stage(xf, None, 0)
    q2, r1 = stage(xf, q1, 1)
    q3, r2 = stage(r1, q2, 2)
    q4, r3 = stage(r2, q3, 3, x_for_partial=None)
    out = _fin(xf, r3, q4)
    return out.reshape(B, T, D)


# speedup vs baseline: 2.0547x; 1.2152x over previous
import functools

import jax
import jax.numpy as jnp
from jax import lax
from jax.experimental import pallas as pl
from jax.experimental.pallas import tpu as pltpu
from jax.experimental.pallas import tpu_sc as plsc

B, T, D = 32, 576, 256
N = B * T
K = 8192
TM = 512
NT = N // TM
KB = 2048

NC, NS = 2, 16
NW = NC * NS
ROWS_W = N // NW
CH = 96
NCH = ROWS_W // CH


def _tc_stage(r_prev, q_prev, emb4, b_row, stage_k_off, x_for_partial):
    with_prev = q_prev is not None
    with_partial = x_for_partial is not None
    emit_r = with_prev or with_partial

    def body(*refs):
        i = 0
        r_ref = refs[i]; i += 1
        if with_prev:
            q_ref = refs[i]; i += 1
        e_ref = refs[i]; i += 1
        b_ref = refs[i]; i += 1
        if with_partial:
            x_ref = refs[i]; i += 1
        codes_ref = refs[i]; i += 1
        if emit_r:
            rout_ref = refs[i]; i += 1

        r = r_ref[...]
        if with_prev:
            r = r - q_ref[...]
        rowsq = jnp.sum(r * r, axis=1, keepdims=True)
        rm2 = r * (-2.0)
        G = 128
        RB = 64
        m_blocks = []
        i_blocks = []
        for kb in range(K // KB):
            e_blk = e_ref[0, kb * KB:(kb + 1) * KB, :]
            mm = lax.dot_general(rm2, e_blk, (((1,), (1,)), ((), ())),
                                 preferred_element_type=jnp.float32)
            for rb in range(TM // RB):
                r0, r1 = rb * RB, (rb + 1) * RB
                rsq = rowsq[r0:r1, :]
                if kb == 0:
                    m_run = (rsq + mm[r0:r1, 0:G]) + b_ref[:, 0:G]
                    i_run = jnp.zeros((RB, G), jnp.float32)
                    m_blocks.append(m_run)
                    i_blocks.append(i_run)
                g0 = 1 if kb == 0 else 0
                m_run, i_run = m_blocks[rb], i_blocks[rb]
                for g in range(g0, KB // G):
                    gg = kb * (KB // G) + g
                    d2g = ((rsq + mm[r0:r1, g * G:(g + 1) * G])
                           + b_ref[:, gg * G:(gg + 1) * G])
                    upd = d2g < m_run
                    m_run = jnp.where(upd, d2g, m_run)
                    i_run = jnp.where(upd, jnp.float32(gg), i_run)
                m_blocks[rb], i_blocks[rb] = m_run, i_run
        m_all = jnp.concatenate(m_blocks, axis=0)
        i_all = jnp.concatenate(i_blocks, axis=0)
        lanef = lax.broadcasted_iota(jnp.int32, (TM, G), 1).astype(
            jnp.float32)
        mins = jnp.min(m_all, axis=1, keepdims=True)
        idxf = jnp.min(jnp.where(m_all == mins, i_all * G + lanef,
                                 jnp.inf), axis=1)
        idx = idxf.astype(jnp.int32) + (stage_k_off * K)
        codes_ref[...] = idx.reshape(1, 1, TM)
        if with_partial:
            rout_ref[...] = (1.0 + x_ref[...]) - r
        elif emit_r:
            rout_ref[...] = r

    row_spec = pl.BlockSpec((TM, D), lambda i: (i, 0))
    in_specs = [row_spec]
    inputs = [r_prev]
    if with_prev:
        in_specs.append(row_spec)
        inputs.append(q_prev)
    in_specs.append(pl.BlockSpec((1, K, D),
                                 lambda i, s=stage_k_off: (s, 0, 0)))
    inputs.append(emb4)
    in_specs.append(pl.BlockSpec((1, K), lambda i: (0, 0)))
    inputs.append(b_row)
    if with_partial:
        in_specs.append(row_spec)
        inputs.append(x_for_partial)

    out_shapes = [jax.ShapeDtypeStruct((NT, 1, TM), jnp.int32)]
    out_specs = [pl.BlockSpec((1, 1, TM), lambda i: (i, 0, 0))]
    if emit_r:
        out_shapes.append(jax.ShapeDtypeStruct((N, D), jnp.float32))
        out_specs.append(row_spec)

    out = pl.pallas_call(
        body,
        grid=(NT,),
        in_specs=in_specs,
        out_specs=out_specs,
        out_shape=out_shapes,
        compiler_params=pltpu.CompilerParams(
            dimension_semantics=("parallel",)),
    )(*inputs)
    return out if emit_r else (out[0], None)


@functools.partial(
    pl.kernel,
    mesh=plsc.VectorSubcoreMesh(core_axis_name="c", subcore_axis_name="s"),
    out_type=jax.ShapeDtypeStruct((N, D), jnp.float32),
    scratch_types=[
        pltpu.VMEM((NCH, CH), jnp.int32),
        pltpu.VMEM((CH, D), jnp.float32),
        pltpu.SemaphoreType.DMA,
    ],
)
def _sc_gather(emb_hbm, codes_hbm, q_hbm, idx_v, rows_v, sem):
    wid = lax.axis_index("s") * NC + lax.axis_index("c")
    pltpu.sync_copy(codes_hbm.at[wid], idx_v)
    base = wid * ROWS_W
    for c in range(NCH):
        pltpu.async_copy(emb_hbm.at[idx_v.at[c]], rows_v, sem).wait()
        pltpu.sync_copy(rows_v, q_hbm.at[pl.ds(base + c * CH, CH)])


def _fin(x, r3, q4):
    def body(x_ref, r_ref, q_ref, o_ref):
        o_ref[...] = (1.0 + x_ref[...]) - (r_ref[...] - q_ref[...])

    spec = pl.BlockSpec((TM, D), lambda i: (i, 0))
    return pl.pallas_call(
        body,
        grid=(NT,),
        in_specs=[spec, spec, spec],
        out_specs=spec,
        out_shape=jax.ShapeDtypeStruct((N, D), jnp.float32),
        compiler_params=pltpu.CompilerParams(
            dimension_semantics=("parallel",)),
    )(x, r3, q4)


def kernel(x, codebooks):
    xf = x.reshape(N, D)
    emb_flat = codebooks.reshape(4 * K, D)
    bnorm = jnp.sum(codebooks * codebooks, axis=-1)

    def stage(r_prev, q_prev, s, x_for_partial=None):
        codes, r_out = _tc_stage(r_prev, q_prev, codebooks,
                                 bnorm[s].reshape(1, K), s, x_for_partial)
        codes32 = codes.reshape(NW, NCH, CH)
        q = _sc_gather(emb_flat, codes32)
        return q, r_out

    q1, _ = stage(xf, None, 0)
    q2, r1 = stage(xf, q1, 1)
    q3, r2 = stage(r1, q2, 2)
    q4, r3 = stage(r2, q3, 3, x_for_partial=None)
    out = _fin(xf, r3, q4)
    return out.reshape(B, T, D)


# speedup vs baseline: 2.1404x; 1.0417x over previous
import functools

import jax
import jax.numpy as jnp
from jax import lax
from jax.experimental import pallas as pl
from jax.experimental.pallas import tpu as pltpu
from jax.experimental.pallas import tpu_sc as plsc

B, T, D = 32, 576, 256
N = B * T
K = 8192
TM = 512
NT = N // TM
KB = 2048

NC, NS = 2, 16
NW = NC * NS
ROWS_W = N // NW
CH = 96
NCH = ROWS_W // CH


def _tc_stage(r_prev, q_prev, emb4, b_row, stage_k_off, x_for_partial):
    with_prev = q_prev is not None
    with_partial = x_for_partial is not None
    emit_r = with_prev or with_partial

    def body(*refs):
        i = 0
        r_ref = refs[i]; i += 1
        if with_prev:
            q_ref = refs[i]; i += 1
        e_ref = refs[i]; i += 1
        b_ref = refs[i]; i += 1
        if with_partial:
            x_ref = refs[i]; i += 1
        codes_ref = refs[i]; i += 1
        if emit_r:
            rout_ref = refs[i]; i += 1

        r = r_ref[...]
        if with_prev:
            r = r - q_ref[...]
        rowsq = jnp.sum(r * r, axis=1, keepdims=True)
        rm2 = r * (-2.0)
        G = 128
        RB = 64
        m_blocks = []
        i_blocks = []
        for kb in range(K // KB):
            e_blk = e_ref[0, kb * KB:(kb + 1) * KB, :]
            mm = lax.dot_general(rm2, e_blk, (((1,), (1,)), ((), ())),
                                 preferred_element_type=jnp.float32)
            for rb in range(TM // RB):
                r0, r1 = rb * RB, (rb + 1) * RB
                rsq = rowsq[r0:r1, :]
                if kb == 0:
                    m_run = (rsq + mm[r0:r1, 0:G]) + b_ref[:, 0:G]
                    i_run = jnp.zeros((RB, G), jnp.float32)
                    m_blocks.append(m_run)
                    i_blocks.append(i_run)
                g0 = 1 if kb == 0 else 0
                m_run, i_run = m_blocks[rb], i_blocks[rb]
                for g in range(g0, KB // G):
                    gg = kb * (KB // G) + g
                    d2g = ((rsq + mm[r0:r1, g * G:(g + 1) * G])
                           + b_ref[:, gg * G:(gg + 1) * G])
                    upd = d2g < m_run
                    m_run = jnp.where(upd, d2g, m_run)
                    i_run = jnp.where(upd, jnp.float32(gg), i_run)
                m_blocks[rb], i_blocks[rb] = m_run, i_run
        m_all = jnp.concatenate(m_blocks, axis=0)
        i_all = jnp.concatenate(i_blocks, axis=0)
        lanef = lax.broadcasted_iota(jnp.int32, (TM, G), 1).astype(
            jnp.float32)
        mins = jnp.min(m_all, axis=1, keepdims=True)
        idxf = jnp.min(jnp.where(m_all == mins, i_all * G + lanef,
                                 jnp.inf), axis=1)
        idx = idxf.astype(jnp.int32) + (stage_k_off * K)
        codes_ref[...] = idx.reshape(1, 1, TM)
        if with_partial:
            rout_ref[...] = (1.0 + x_ref[...]) - r
        elif emit_r:
            rout_ref[...] = r

    row_spec = pl.BlockSpec((TM, D), lambda i: (i, 0))
    in_specs = [row_spec]
    inputs = [r_prev]
    if with_prev:
        in_specs.append(row_spec)
        inputs.append(q_prev)
    in_specs.append(pl.BlockSpec((1, K, D),
                                 lambda i, s=stage_k_off: (s, 0, 0)))
    inputs.append(emb4)
    in_specs.append(pl.BlockSpec((1, K), lambda i: (0, 0)))
    inputs.append(b_row)
    if with_partial:
        in_specs.append(row_spec)
        inputs.append(x_for_partial)

    n = r_prev.shape[0]
    nt = n // TM
    out_shapes = [jax.ShapeDtypeStruct((nt, 1, TM), jnp.int32)]
    out_specs = [pl.BlockSpec((1, 1, TM), lambda i: (i, 0, 0))]
    if emit_r:
        out_shapes.append(jax.ShapeDtypeStruct((n, D), jnp.float32))
        out_specs.append(row_spec)

    out = pl.pallas_call(
        body,
        grid=(nt,),
        in_specs=in_specs,
        out_specs=out_specs,
        out_shape=out_shapes,
        compiler_params=pltpu.CompilerParams(
            dimension_semantics=("parallel",)),
    )(*inputs)
    return out if emit_r else (out[0], None)


_SC_CACHE = {}


def _sc_gather(n_rows):
    if n_rows in _SC_CACHE:
        return _SC_CACHE[n_rows]
    rows_w = n_rows // NW
    nch = rows_w // CH

    @functools.partial(
        pl.kernel,
        mesh=plsc.VectorSubcoreMesh(core_axis_name="c",
                                    subcore_axis_name="s"),
        out_type=jax.ShapeDtypeStruct((n_rows, D), jnp.float32),
        scratch_types=[
            pltpu.VMEM((nch, CH), jnp.int32),
            pltpu.VMEM((CH, D), jnp.float32),
            pltpu.SemaphoreType.DMA,
        ],
    )
    def gather(emb_hbm, codes_hbm, q_hbm, idx_v, rows_v, sem):
        wid = lax.axis_index("s") * NC + lax.axis_index("c")
        pltpu.sync_copy(codes_hbm.at[wid], idx_v)
        base = wid * rows_w
        for c in range(nch):
            pltpu.async_copy(emb_hbm.at[idx_v.at[c]], rows_v, sem).wait()
            pltpu.sync_copy(rows_v, q_hbm.at[pl.ds(base + c * CH, CH)])

    _SC_CACHE[n_rows] = gather
    return gather


def _fin(x, r3, q4):
    n = x.shape[0]
    def body(x_ref, r_ref, q_ref, o_ref):
        o_ref[...] = (1.0 + x_ref[...]) - (r_ref[...] - q_ref[...])

    spec = pl.BlockSpec((TM, D), lambda i: (i, 0))
    return pl.pallas_call(
        body,
        grid=(n // TM,),
        in_specs=[spec, spec, spec],
        out_specs=spec,
        out_shape=jax.ShapeDtypeStruct((n, D), jnp.float32),
        compiler_params=pltpu.CompilerParams(
            dimension_semantics=("parallel",)),
    )(x, r3, q4)


def kernel(x, codebooks):
    xf = x.reshape(N, D)
    emb_flat = codebooks.reshape(4 * K, D)
    bnorm = jnp.sum(codebooks * codebooks, axis=-1)
    brows = [bnorm[s].reshape(1, K) for s in range(4)]

    H = N // 2
    halves = [xf[:H], xf[H:]]
    sc = _sc_gather(H)
    nch_h = (H // NW) // CH

    def tc(r_prev, q_prev, s):
        codes, r_out = _tc_stage(r_prev, q_prev, codebooks, brows[s],
                                 s, None)
        return codes.reshape(NW, nch_h, CH), r_out

    q = [None, None]
    r = list(halves)
    out = [None, None]
    codes = [None, None]
    for s in range(4):
        for h in range(2):
            if s == 0:
                codes[h], _ = tc(r[h], None, 0)
            else:
                codes[h], r[h] = tc(r[h], q[h], s)
        for h in range(2):
            q[h] = sc(emb_flat, codes[h])
    for h in range(2):
        out[h] = _fin(halves[h], r[h], q[h])
    return jnp.concatenate(out, axis=0).reshape(B, T, D)
